# baseline (speedup 1.0000x reference)
import functools

import jax
import jax.numpy as jnp
from jax import lax
from jax.experimental import pallas as pl
from jax.experimental.pallas import tpu as pltpu
from jax.experimental.pallas import tpu_sc as plsc

NC = 2
NS = 16
L = 16
NW = NC * NS

D = 32
H = 4
T = 50
TH = T * H
DT = D // 8
BR = 128
TP = BR + 1
NB = 4


def _body(ids_hbm, offb_hbm, table_hbm, out_hbm, idx_v, off_v, rows_v,
          trow_v, gsem0, gsem1, gsem2, gsem3, osem0, osem1, osem2, osem3):
    bpw = ids_hbm.shape[2] // NW
    wid = lax.axis_index("s") * NC + lax.axis_index("c")
    b0 = wid * bpw

    pltpu.sync_copy(ids_hbm.at[:, :, pl.ds(b0, bpw)], idx_v)
    pltpu.sync_copy(offb_hbm, off_v)

    def add_th(th, _):
        t = lax.shift_right_logical(th, 2)
        h = lax.bitwise_and(th, H - 1)
        off = off_v[h]
        for g in range(bpw // L):
            sl = pl.ds(g * L, L)
            idx_v[t, h, sl] = idx_v[t, h, sl] + off
        return 0

    lax.fori_loop(0, TH, add_th, 0)

    gsems = (gsem0, gsem1, gsem2, gsem3)
    osems = (osem0, osem1, osem2, osem3)

    def fire_gather(th, buf):
        t = lax.shift_right_logical(th, 2)
        h = lax.bitwise_and(th, H - 1)
        pltpu.async_copy(table_hbm.at[idx_v.at[t, h]], rows_v.at[buf],
                         gsems[buf])

    def wait_out(buf):
        for dt in range(DT):
            pltpu.make_async_copy(
                trow_v.at[buf, pl.ds(dt * 8, 8), pl.ds(0, BR)],
                out_hbm.at[0, 0, dt, 0],
                osems[buf],
            ).wait()

    for buf in range(NB):
        fire_gather(buf, buf)

    iota = lax.iota(jnp.int32, L)
    ib = [iota + g * L for g in range(D // L)]

    def stepn(i4, _):
        for buf in range(NB):
            th = i4 * NB + buf
            t = lax.shift_right_logical(th, 2)
            h = lax.bitwise_and(th, H - 1)

            pltpu.make_async_copy(
                table_hbm.at[idx_v.at[0, 0]], rows_v.at[buf], gsems[buf]
            ).wait()

            @pl.when(th >= NB)
            def _():
                wait_out(buf)

            tbuf = trow_v.at[buf]
            for b in range(bpw):
                bvec = jnp.full((L,), b, jnp.int32)
                for dg in range(D // L):
                    vals = rows_v[buf, b, pl.ds(dg * L, L)]
                    plsc.store_scatter(tbuf, [ib[dg], bvec], vals)

            for dt in range(DT):
                pltpu.async_copy(
                    trow_v.at[buf, pl.ds(dt * 8, 8), pl.ds(0, BR)],
                    out_hbm.at[t, h, dt, wid],
                    osems[buf],
                )

            @pl.when(th + NB < TH)
            def _():
                fire_gather(th + NB, buf)
        return 0

    lax.fori_loop(0, TH // NB, stepn, 0)

    for buf in range(NB):
        wait_out(buf)


def kernel(input_ids, table, offsets):
    B = input_ids.shape[0]
    bpw = B // NW

    ids_t = jnp.transpose(input_ids.astype(jnp.int32), (1, 2, 0))
    offb = jnp.broadcast_to(offsets.astype(jnp.int32)[:, None], (H, L))
    tab = table.astype(jnp.float32)

    run = functools.partial(
        pl.kernel,
        mesh=plsc.VectorSubcoreMesh(core_axis_name="c", subcore_axis_name="s"),
        out_type=jax.ShapeDtypeStruct((T, H, DT, B // BR, 8, BR), jnp.float32),
        scratch_types=[
            pltpu.VMEM((T, H, bpw), jnp.int32),
            pltpu.VMEM((H, L), jnp.int32),
            pltpu.VMEM((NB, bpw, D), jnp.float32),
            pltpu.VMEM((NB, D, TP), jnp.float32),
        ] + [pltpu.SemaphoreType.DMA] * (2 * NB),
        compiler_params=pltpu.CompilerParams(
            use_tc_tiling_on_sc=False, needs_layout_passes=False
        ),
    )(_body)

    o6 = run(ids_t, offb, tab)
    out = jnp.transpose(o6, (3, 5, 0, 1, 2, 4))
    return out.reshape(B, T, H, D)


# speedup vs baseline: 1.2218x; 1.2218x over previous
import functools

import jax
import jax.numpy as jnp
from jax import lax
from jax.experimental import pallas as pl
from jax.experimental.pallas import tpu as pltpu
from jax.experimental.pallas import tpu_sc as plsc

NC = 2
NS = 16
L = 16
NW = NC * NS

D = 32
H = 4
T = 50
TH = T * H
DT = D // 8
BR = 128
TP = BR + 1


def _body(ids_hbm, offb_hbm, table_hbm, out_hbm,
          idx_v, off_v, rows_v, trow_v, gsem0, gsem1, osem0, osem1):
    bpw = ids_hbm.shape[2] // NW
    wid = lax.axis_index("s") * NC + lax.axis_index("c")
    b0 = wid * bpw

    pltpu.sync_copy(ids_hbm.at[:, :, pl.ds(b0, bpw)], idx_v)
    pltpu.sync_copy(offb_hbm, off_v)

    def add_th(th, _):
        t = lax.shift_right_logical(th, 2)
        h = lax.bitwise_and(th, H - 1)
        off = off_v[h]
        for g in range(bpw // L):
            sl = pl.ds(g * L, L)
            idx_v[t, h, sl] = idx_v[t, h, sl] + off
        return 0

    lax.fori_loop(0, TH, add_th, 0)

    gsems = (gsem0, gsem1)
    osems = (osem0, osem1)

    def fire_gather(th, buf):
        t = lax.shift_right_logical(th, 2)
        h = lax.bitwise_and(th, H - 1)
        pltpu.async_copy(table_hbm.at[idx_v.at[t, h]], rows_v.at[buf],
                         gsems[buf])

    def wait_out(buf):
        for dt in range(DT):
            pltpu.make_async_copy(
                trow_v.at[buf, pl.ds(dt * 8, 8), pl.ds(0, BR)],
                out_hbm.at[0, 0, dt, 0],
                osems[buf],
            ).wait()

    fire_gather(0, 0)
    fire_gather(1, 1)

    iota = lax.iota(jnp.int32, L)
    ib = [iota + g * L for g in range(D // L)]

    def step2(i2, _):
        for buf in range(2):
            th = i2 * 2 + buf
            t = lax.shift_right_logical(th, 2)
            h = lax.bitwise_and(th, H - 1)

            pltpu.make_async_copy(
                table_hbm.at[idx_v.at[0, 0]], rows_v.at[buf], gsems[buf]
            ).wait()

            @pl.when(th >= 2)
            def _():
                wait_out(buf)

            tbuf = trow_v.at[buf]
            rbuf = rows_v.at[buf]

            @plsc.parallel_loop(0, bpw, unroll=8)
            def _(b):
                bvec = jnp.full((L,), b, jnp.int32)
                for dg in range(D // L):
                    vals = rbuf[b, pl.ds(dg * L, L)]
                    plsc.store_scatter(tbuf, [ib[dg], bvec], vals)

            for dt in range(DT):
                pltpu.async_copy(
                    trow_v.at[buf, pl.ds(dt * 8, 8), pl.ds(0, BR)],
                    out_hbm.at[t, h, dt, wid],
                    osems[buf],
                )

            @pl.when(th + 2 < TH)
            def _():
                fire_gather(th + 2, buf)
        return 0

    lax.fori_loop(0, TH // 2, step2, 0)

    for buf in range(2):
        wait_out(buf)


def kernel(input_ids, table, offsets):
    B = input_ids.shape[0]
    bpw = B // NW

    ids_t = jnp.transpose(input_ids.astype(jnp.int32), (1, 2, 0))
    offb = jnp.broadcast_to(offsets.astype(jnp.int32)[:, None], (H, L))
    tab = table.astype(jnp.float32)

    run = functools.partial(
        pl.kernel,
        mesh=plsc.VectorSubcoreMesh(core_axis_name="c", subcore_axis_name="s"),
        out_type=jax.ShapeDtypeStruct((T, H, DT, B // BR, 8, BR), jnp.float32),
        scratch_types=[
            pltpu.VMEM((T, H, bpw), jnp.int32),
            pltpu.VMEM((H, L), jnp.int32),
            pltpu.VMEM((2, bpw, D), jnp.float32),
            pltpu.VMEM((2, D, TP), jnp.float32),
            pltpu.SemaphoreType.DMA,
            pltpu.SemaphoreType.DMA,
            pltpu.SemaphoreType.DMA,
            pltpu.SemaphoreType.DMA,
        ],
        compiler_params=pltpu.CompilerParams(
            use_tc_tiling_on_sc=False, needs_layout_passes=False
        ),
    )(_body)

    o6 = run(ids_t, offb, tab)
    out = jnp.transpose(o6, (3, 5, 0, 1, 2, 4))
    return out.reshape(B, T, H, D)


# speedup vs baseline: 1.3190x; 1.0795x over previous
import functools

import jax
import jax.numpy as jnp
from jax import lax
from jax.experimental import pallas as pl
from jax.experimental.pallas import tpu as pltpu
from jax.experimental.pallas import tpu_sc as plsc

NC = 2
NS = 16
L = 16
NW = NC * NS

D = 32
H = 4
T = 50
TH = T * H
DT = D // 8
BR = 128
TP = BR + 1


NB = 4


def _body(ids_hbm, offb_hbm, table_hbm, out_hbm, idx_v, off_v, rows_v,
          trow_v, gsem0, gsem1, gsem2, gsem3, osem0, osem1, osem2, osem3):
    bpw = ids_hbm.shape[2] // NW
    wid = lax.axis_index("s") * NC + lax.axis_index("c")
    b0 = wid * bpw

    pltpu.sync_copy(ids_hbm.at[:, :, pl.ds(b0, bpw)], idx_v)
    pltpu.sync_copy(offb_hbm, off_v)

    @plsc.parallel_loop(0, TH, unroll=4)
    def _(th):
        t = lax.shift_right_logical(th, 2)
        h = lax.bitwise_and(th, H - 1)
        off = off_v[h]
        for g in range(bpw // L):
            sl = pl.ds(g * L, L)
            idx_v[t, h, sl] = idx_v[t, h, sl] + off

    gsems = (gsem0, gsem1, gsem2, gsem3)
    osems = (osem0, osem1, osem2, osem3)

    def fire_gather(th, buf):
        t = lax.shift_right_logical(th, 2)
        h = lax.bitwise_and(th, H - 1)
        pltpu.async_copy(table_hbm.at[idx_v.at[t, h]], rows_v.at[buf],
                         gsems[buf])

    def wait_out(buf):
        for dt in range(DT):
            pltpu.make_async_copy(
                trow_v.at[buf, pl.ds(dt * 8, 8), pl.ds(0, BR)],
                out_hbm.at[0, 0, dt, 0],
                osems[buf],
            ).wait()

    for pbuf in range(NB):
        fire_gather(pbuf, pbuf)

    iota = lax.iota(jnp.int32, L)
    ib = [iota + g * L for g in range(D // L)]

    def step2(i2, _):
        for buf in range(NB):
            th = i2 * NB + buf
            t = lax.shift_right_logical(th, 2)
            h = lax.bitwise_and(th, H - 1)

            pltpu.make_async_copy(
                table_hbm.at[idx_v.at[0, 0]], rows_v.at[buf], gsems[buf]
            ).wait()

            @pl.when(th >= NB)
            def _():
                wait_out(buf)

            tbuf = trow_v.at[buf]
            rbuf = rows_v.at[buf]

            @plsc.parallel_loop(0, bpw, unroll=8)
            def _(b):
                bvec = jnp.full((L,), b, jnp.int32)
                for dg in range(D // L):
                    vals = rbuf[b, pl.ds(dg * L, L)]
                    plsc.store_scatter(tbuf, [ib[dg], bvec], vals)

            for dt in range(DT):
                pltpu.async_copy(
                    trow_v.at[buf, pl.ds(dt * 8, 8), pl.ds(0, BR)],
                    out_hbm.at[t, h, dt, wid],
                    osems[buf],
                )

            @pl.when(th + NB < TH)
            def _():
                fire_gather(th + NB, buf)
        return 0

    lax.fori_loop(0, TH // NB, step2, 0)

    for buf in range(NB):
        wait_out(buf)


def kernel(input_ids, table, offsets):
    B = input_ids.shape[0]
    bpw = B // NW

    ids_t = jnp.transpose(input_ids.astype(jnp.int32), (1, 2, 0))
    offb = jnp.broadcast_to(offsets.astype(jnp.int32)[:, None], (H, L))
    tab = table.astype(jnp.float32)

    run = functools.partial(
        pl.kernel,
        mesh=plsc.VectorSubcoreMesh(core_axis_name="c", subcore_axis_name="s"),
        out_type=jax.ShapeDtypeStruct((T, H, DT, B // BR, 8, BR), jnp.float32),
        scratch_types=[
            pltpu.VMEM((T, H, bpw), jnp.int32),
            pltpu.VMEM((H, L), jnp.int32),
            pltpu.VMEM((NB, bpw, D), jnp.float32),
            pltpu.VMEM((NB, D, TP), jnp.float32),
        ] + [pltpu.SemaphoreType.DMA] * (2 * NB),
        compiler_params=pltpu.CompilerParams(
            use_tc_tiling_on_sc=False, needs_layout_passes=False
        ),
    )(_body)

    o6 = run(ids_t, offb, tab)
    out = jnp.transpose(o6, (3, 5, 0, 1, 2, 4))
    return out.reshape(B, T, H, D)


# speedup vs baseline: 1.3326x; 1.0103x over previous
import functools

import jax
import jax.numpy as jnp
from jax import lax
from jax.experimental import pallas as pl
from jax.experimental.pallas import tpu as pltpu
from jax.experimental.pallas import tpu_sc as plsc

NC = 2
NS = 16
L = 16
NW = NC * NS

D = 32
H = 4
T = 50
TH = T * H
DT = D // 8
BR = 128
TP = BR + 1


NB = 8


def _body(ids_hbm, offb_hbm, table_hbm, out_hbm, idx_v, off_v, rows_v,
          trow_v, gsem0, gsem1, gsem2, gsem3, gsem4, gsem5, gsem6, gsem7,
          osem0, osem1, osem2, osem3, osem4, osem5, osem6, osem7):
    bpw = ids_hbm.shape[2] // NW
    wid = lax.axis_index("s") * NC + lax.axis_index("c")
    b0 = wid * bpw

    pltpu.sync_copy(ids_hbm.at[:, :, pl.ds(b0, bpw)], idx_v)
    pltpu.sync_copy(offb_hbm, off_v)

    @plsc.parallel_loop(0, TH, unroll=4)
    def _(th):
        t = lax.shift_right_logical(th, 2)
        h = lax.bitwise_and(th, H - 1)
        off = off_v[h]
        for g in range(bpw // L):
            sl = pl.ds(g * L, L)
            idx_v[t, h, sl] = idx_v[t, h, sl] + off

    gsems = (gsem0, gsem1, gsem2, gsem3, gsem4, gsem5, gsem6, gsem7)
    osems = (osem0, osem1, osem2, osem3, osem4, osem5, osem6, osem7)

    def fire_gather(th, buf):
        t = lax.shift_right_logical(th, 2)
        h = lax.bitwise_and(th, H - 1)
        pltpu.async_copy(table_hbm.at[idx_v.at[t, h]], rows_v.at[buf],
                         gsems[buf])

    def wait_out(buf):
        for dt in range(DT):
            pltpu.make_async_copy(
                trow_v.at[buf, pl.ds(dt * 8, 8), pl.ds(0, BR)],
                out_hbm.at[0, 0, dt, 0],
                osems[buf],
            ).wait()

    for pbuf in range(NB):
        fire_gather(pbuf, pbuf)

    iota = lax.iota(jnp.int32, L)
    ib = [iota + g * L for g in range(D // L)]

    def step2(i2, _):
        for buf in range(NB):
            th = i2 * NB + buf
            t = lax.shift_right_logical(th, 2)
            h = lax.bitwise_and(th, H - 1)

            pltpu.make_async_copy(
                table_hbm.at[idx_v.at[0, 0]], rows_v.at[buf], gsems[buf]
            ).wait()

            @pl.when(th >= NB)
            def _():
                wait_out(buf)

            tbuf = trow_v.at[buf]
            rbuf = rows_v.at[buf]

            @plsc.parallel_loop(0, bpw, unroll=8)
            def _(b):
                bvec = jnp.full((L,), b, jnp.int32)
                for dg in range(D // L):
                    vals = rbuf[b, pl.ds(dg * L, L)]
                    plsc.store_scatter(tbuf, [ib[dg], bvec], vals)

            for dt in range(DT):
                pltpu.async_copy(
                    trow_v.at[buf, pl.ds(dt * 8, 8), pl.ds(0, BR)],
                    out_hbm.at[t, h, dt, wid],
                    osems[buf],
                )

            @pl.when(th + NB < TH)
            def _():
                fire_gather(th + NB, buf)
        return 0

    lax.fori_loop(0, TH // NB, step2, 0)

    for buf in range(NB):
        wait_out(buf)


def kernel(input_ids, table, offsets):
    B = input_ids.shape[0]
    bpw = B // NW

    ids_t = jnp.transpose(input_ids.astype(jnp.int32), (1, 2, 0))
    offb = jnp.broadcast_to(offsets.astype(jnp.int32)[:, None], (H, L))
    tab = table.astype(jnp.float32)

    run = functools.partial(
        pl.kernel,
        mesh=plsc.VectorSubcoreMesh(core_axis_name="c", subcore_axis_name="s"),
        out_type=jax.ShapeDtypeStruct((T, H, DT, B // BR, 8, BR), jnp.float32),
        scratch_types=[
            pltpu.VMEM((T, H, bpw), jnp.int32),
            pltpu.VMEM((H, L), jnp.int32),
            pltpu.VMEM((NB, bpw, D), jnp.float32),
            pltpu.VMEM((NB, D, TP), jnp.float32),
        ] + [pltpu.SemaphoreType.DMA] * (2 * NB),
        compiler_params=pltpu.CompilerParams(
            use_tc_tiling_on_sc=False, needs_layout_passes=False
        ),
    )(_body)

    o6 = run(ids_t, offb, tab)
    out = jnp.transpose(o6, (3, 5, 0, 1, 2, 4))
    return out.reshape(B, T, H, D)
